# Initial kernel scaffold; baseline (speedup 1.0000x reference)
import jax
import jax.numpy as jnp
from jax.experimental import pallas as pl


def kernel(x, edge_index, Wl0, bl0, Wr0, gamma0, beta0, Wl1, bl1, Wr1, gamma1, beta1, Wl2, bl2, Wr2, gamma2, beta2, Wc1, bc1, Wc2, bc2):
    raise NotImplementedError("write your pallas kernel here")


# speedup vs baseline: 4.4860x; 4.4860x over previous
import functools

import jax
import jax.numpy as jnp
from jax import lax
from jax.experimental import pallas as pl
from jax.experimental.pallas import tpu as pltpu
from jax.experimental.pallas import tpu_sc as plsc

N = 100000
E = 3200000
H = 64
EPS = 1e-5

NC = 2
NS = 16
LANES = 16

EC = 256
NCHUNK = E // EC
RANGE = 25600
NRANGE = 4
NPAD = RANGE * NRANGE
TRASH = RANGE

BT = 2000
GRID = N // BT


def _agg16_kernel(x_hbm, src_hbm, dst_hbm, out_hbm, srcb, dstb, rows, acc,
                  sem):
    c = lax.axis_index("c")
    s = lax.axis_index("s")

    def _zero_rows(i, _):
        rows[i, pl.ds(0, 16)] = jnp.zeros((16,), jnp.float32)
        return 0
    lax.fori_loop(0, EC, _zero_rows, 0)

    def _zero_acc(j, _):
        pltpu.sync_copy(rows, acc.at[pl.ds((s * 25 + j) * EC, EC)])
        return 0
    lax.fori_loop(0, 25, _zero_acc, 0)
    plsc.subcore_barrier()

    half = NCHUNK // NC
    per = half // NS
    rem = half - per * NS
    nch = jnp.where(s < rem, per + 1, per)

    def _chunk(i, _):
        ci = c * half + s + i * NS
        pltpu.sync_copy(src_hbm.at[ci], srcb)
        pltpu.sync_copy(dst_hbm.at[ci], dstb)
        descs = []
        for k in range(EC // 128):
            descs.append(pltpu.async_copy(
                x_hbm.at[srcb.at[k]], rows.at[pl.ds(k * 128, 128)], sem))
        for d in descs:
            d.wait()
        for k in range(EC // 128):
            pltpu.sync_copy(rows.at[pl.ds(k * 128, 128)],
                            acc.at[dstb.at[k]], add=True)
        return 0
    lax.fori_loop(0, nch, _chunk, 0)
    plsc.subcore_barrier()

    def _out(j, _):
        r0 = (s * 25 + j) * EC
        pltpu.sync_copy(acc.at[pl.ds(r0, EC)], out_hbm.at[c, pl.ds(r0, EC)])
        return 0
    lax.fori_loop(0, 25, _out, 0)


def _agg16(x_pad, src3, dst3):
    mesh = plsc.VectorSubcoreMesh(core_axis_name="c", subcore_axis_name="s", num_cores=NC, num_subcores=NS)
    f = pl.kernel(
        _agg16_kernel,
        out_type=jax.ShapeDtypeStruct((NC, NPAD, 16), jnp.float32),
        mesh=mesh,
        compiler_params=pltpu.CompilerParams(use_tc_tiling_on_sc=False),
        scratch_types=[
            pltpu.VMEM((EC // 128, 128), jnp.int32),
            pltpu.VMEM((EC // 128, 128), jnp.int32),
            pltpu.VMEM((EC, 16), jnp.float32),
            pltpu.VMEM_SHARED((NPAD, 16), jnp.float32),
            pltpu.SemaphoreType.DMA,
        ],
    )
    return f(x_pad, src3, dst3)


def _agg64_kernel(h_hbm, src_hbm, dst_hbm, out_hbm, srcb, dstb, dstlb, rows,
                  acc, sem):
    c = lax.axis_index("c")
    s = lax.axis_index("s")

    per = NCHUNK // NS
    rem = NCHUNK - per * NS
    nch = jnp.where(s < rem, per + 1, per)

    for rr in range(2):
        rng = c * 2 + rr
        base = rng * RANGE

        def _zero_rows(i, _):
            for col in range(4):
                rows[i, pl.ds(col * 16, 16)] = jnp.zeros((16,), jnp.float32)
            return 0
        lax.fori_loop(0, EC, _zero_rows, 0)

        def _zero_acc(j, _):
            pltpu.sync_copy(rows.at[pl.ds(0, 200)],
                            acc.at[pl.ds((s * 8 + j) * 200, 200)])
            return 0
        lax.fori_loop(0, 8, _zero_acc, 0)

        @pl.when(s == 0)
        def _zero_trash():
            pltpu.sync_copy(rows.at[pl.ds(0, 8)], acc.at[pl.ds(RANGE, 8)])
        plsc.subcore_barrier()

        def _chunk(i, _):
            ci = s + i * NS
            pltpu.sync_copy(src_hbm.at[ci], srcb)
            pltpu.sync_copy(dst_hbm.at[ci], dstb)
            for k in range(EC // 128):
                for j in range(8):
                    v = dstb[k, pl.ds(j * 16, 16)]
                    m = (v >= base) & (v < base + RANGE)
                    dstlb[k, pl.ds(j * 16, 16)] = jnp.where(
                        m, v - base, TRASH)
            descs = []
            for k in range(EC // 128):
                descs.append(pltpu.async_copy(
                    h_hbm.at[srcb.at[k]], rows.at[pl.ds(k * 128, 128)], sem))
            for d in descs:
                d.wait()
            for k in range(EC // 128):
                pltpu.sync_copy(rows.at[pl.ds(k * 128, 128)],
                                acc.at[dstlb.at[k]], add=True)
            return 0
        lax.fori_loop(0, nch, _chunk, 0)
        plsc.subcore_barrier()

        def _out(j, _):
            r0 = (s * 4 + j) * 400
            pltpu.sync_copy(acc.at[pl.ds(r0, 400)],
                            out_hbm.at[pl.ds(base + r0, 400)])
            return 0
        lax.fori_loop(0, 4, _out, 0)
        plsc.subcore_barrier()


def _agg64(h, src3, dst3):
    mesh = plsc.VectorSubcoreMesh(core_axis_name="c", subcore_axis_name="s", num_cores=NC, num_subcores=NS)
    f = pl.kernel(
        _agg64_kernel,
        out_type=jax.ShapeDtypeStruct((NPAD, H), jnp.float32),
        mesh=mesh,
        compiler_params=pltpu.CompilerParams(use_tc_tiling_on_sc=False),
        scratch_types=[
            pltpu.VMEM((EC // 128, 128), jnp.int32),
            pltpu.VMEM((EC // 128, 128), jnp.int32),
            pltpu.VMEM((EC // 128, 128), jnp.int32),
            pltpu.VMEM((EC, H), jnp.float32),
            pltpu.VMEM_SHARED((RANGE + 8, H), jnp.float32),
            pltpu.SemaphoreType.DMA,
        ],
    )
    return f(h, src3, dst3)


def _layer0_tc_kernel(agg_a, agg_b, x, wl, bl, wr, t_ref, sum_ref, sq_ref,
                      rec_ref):
    i = pl.program_id(0)
    agg = agg_a[...] + agg_b[...]
    deg = agg[:, 12:13]
    rec = 1.0 / jnp.maximum(deg, 1.0)
    mean = agg * rec
    t = (jnp.dot(mean, wl[...], preferred_element_type=jnp.float32)
         + bl[...]
         + jnp.dot(x[...], wr[...], preferred_element_type=jnp.float32))
    t_ref[...] = t
    rec_ref[...] = rec
    ps = jnp.sum(t, axis=0, keepdims=True)
    ps2 = jnp.sum(t * t, axis=0, keepdims=True)

    @pl.when(i == 0)
    def _():
        sum_ref[...] = ps
        sq_ref[...] = ps2

    @pl.when(i > 0)
    def _():
        sum_ref[...] += ps
        sq_ref[...] += ps2


def _layer0_tc(agg_a, agg_b, x_pad, wl, bl, wr):
    return pl.pallas_call(
        _layer0_tc_kernel,
        grid=(GRID,),
        in_specs=[
            pl.BlockSpec((BT, 16), lambda i: (i, 0)),
            pl.BlockSpec((BT, 16), lambda i: (i, 0)),
            pl.BlockSpec((BT, 16), lambda i: (i, 0)),
            pl.BlockSpec((16, H), lambda i: (0, 0)),
            pl.BlockSpec((1, H), lambda i: (0, 0)),
            pl.BlockSpec((16, H), lambda i: (0, 0)),
        ],
        out_specs=[
            pl.BlockSpec((BT, H), lambda i: (i, 0)),
            pl.BlockSpec((1, H), lambda i: (0, 0)),
            pl.BlockSpec((1, H), lambda i: (0, 0)),
            pl.BlockSpec((BT, 1), lambda i: (i, 0)),
        ],
        out_shape=[
            jax.ShapeDtypeStruct((N, H), jnp.float32),
            jax.ShapeDtypeStruct((1, H), jnp.float32),
            jax.ShapeDtypeStruct((1, H), jnp.float32),
            jax.ShapeDtypeStruct((N, 1), jnp.float32),
        ],
    )(agg_a, agg_b, x_pad, wl, bl, wr)


def _layer12_tc_kernel(agg, h, rec, wl, bl, wr, t_ref, sum_ref, sq_ref):
    i = pl.program_id(0)
    mean = agg[...] * rec[...]
    t = (jnp.dot(mean, wl[...], preferred_element_type=jnp.float32)
         + bl[...]
         + jnp.dot(h[...], wr[...], preferred_element_type=jnp.float32))
    t_ref[...] = t
    ps = jnp.sum(t, axis=0, keepdims=True)
    ps2 = jnp.sum(t * t, axis=0, keepdims=True)

    @pl.when(i == 0)
    def _():
        sum_ref[...] = ps
        sq_ref[...] = ps2

    @pl.when(i > 0)
    def _():
        sum_ref[...] += ps
        sq_ref[...] += ps2


def _layer12_tc(agg, h, rec, wl, bl, wr):
    return pl.pallas_call(
        _layer12_tc_kernel,
        grid=(GRID,),
        in_specs=[
            pl.BlockSpec((BT, H), lambda i: (i, 0)),
            pl.BlockSpec((BT, H), lambda i: (i, 0)),
            pl.BlockSpec((BT, 1), lambda i: (i, 0)),
            pl.BlockSpec((H, H), lambda i: (0, 0)),
            pl.BlockSpec((1, H), lambda i: (0, 0)),
            pl.BlockSpec((H, H), lambda i: (0, 0)),
        ],
        out_specs=[
            pl.BlockSpec((BT, H), lambda i: (i, 0)),
            pl.BlockSpec((1, H), lambda i: (0, 0)),
            pl.BlockSpec((1, H), lambda i: (0, 0)),
        ],
        out_shape=[
            jax.ShapeDtypeStruct((N, H), jnp.float32),
            jax.ShapeDtypeStruct((1, H), jnp.float32),
            jax.ShapeDtypeStruct((1, H), jnp.float32),
        ],
    )(agg, h, rec, wl, bl, wr)


def _bn_relu_kernel(t, sum_in, sq_in, gamma, beta, out_ref):
    mu = sum_in[...] * (1.0 / N)
    var = sq_in[...] * (1.0 / N) - mu * mu
    sc = gamma[...] * lax.rsqrt(var + EPS)
    sh = beta[...] - mu * sc
    out_ref[...] = jnp.maximum(t[...] * sc + sh, 0.0)


def _bn_relu(t, s, q, gamma, beta):
    return pl.pallas_call(
        _bn_relu_kernel,
        grid=(GRID,),
        in_specs=[
            pl.BlockSpec((BT, H), lambda i: (i, 0)),
            pl.BlockSpec((1, H), lambda i: (0, 0)),
            pl.BlockSpec((1, H), lambda i: (0, 0)),
            pl.BlockSpec((1, H), lambda i: (0, 0)),
            pl.BlockSpec((1, H), lambda i: (0, 0)),
        ],
        out_specs=pl.BlockSpec((BT, H), lambda i: (i, 0)),
        out_shape=jax.ShapeDtypeStruct((N, H), jnp.float32),
    )(t, s, q, gamma, beta)


def _bn_head_kernel(t, sum_in, sq_in, gamma, beta, wc1, bc1, wc2, bc2,
                    out_ref):
    mu = sum_in[...] * (1.0 / N)
    var = sq_in[...] * (1.0 / N) - mu * mu
    sc = gamma[...] * lax.rsqrt(var + EPS)
    sh = beta[...] - mu * sc
    h3 = t[...] * sc + sh
    p = jnp.maximum(
        jnp.dot(h3, wc1[...], preferred_element_type=jnp.float32) + bc1[...],
        0.0)
    out_ref[...] = jnp.sum(p * wc2[...], axis=1, keepdims=True) + bc2[...]


def _bn_head(t, s, q, gamma, beta, wc1, bc1, wc2, bc2):
    return pl.pallas_call(
        _bn_head_kernel,
        grid=(GRID,),
        in_specs=[
            pl.BlockSpec((BT, H), lambda i: (i, 0)),
            pl.BlockSpec((1, H), lambda i: (0, 0)),
            pl.BlockSpec((1, H), lambda i: (0, 0)),
            pl.BlockSpec((1, H), lambda i: (0, 0)),
            pl.BlockSpec((1, H), lambda i: (0, 0)),
            pl.BlockSpec((H, 32), lambda i: (0, 0)),
            pl.BlockSpec((1, 32), lambda i: (0, 0)),
            pl.BlockSpec((1, 32), lambda i: (0, 0)),
            pl.BlockSpec((1, 1), lambda i: (0, 0)),
        ],
        out_specs=pl.BlockSpec((BT, 1), lambda i: (i, 0)),
        out_shape=jax.ShapeDtypeStruct((N, 1), jnp.float32),
    )(t, s, q, gamma, beta, wc1, bc1, wc2, bc2)


def kernel(x, edge_index, Wl0, bl0, Wr0, gamma0, beta0, Wl1, bl1, Wr1,
           gamma1, beta1, Wl2, bl2, Wr2, gamma2, beta2, Wc1, bc1, Wc2, bc2):
    src3 = edge_index[0].reshape(NCHUNK, EC // 128, 128)
    dst3 = edge_index[1].reshape(NCHUNK, EC // 128, 128)

    pad = jnp.concatenate(
        [jnp.ones((N, 1), jnp.float32), jnp.zeros((N, 3), jnp.float32)],
        axis=1)
    x_pad = jnp.concatenate([x, pad], axis=1)

    wl0 = jnp.zeros((16, H), jnp.float32).at[:12].set(Wl0.T)
    wr0 = jnp.zeros((16, H), jnp.float32).at[:12].set(Wr0.T)
    wl1, wr1 = Wl1.T, Wr1.T
    wl2, wr2 = Wl2.T, Wr2.T
    bl0r, bl1r, bl2r = (b.reshape(1, H) for b in (bl0, bl1, bl2))
    g0, g1, g2 = (g.reshape(1, H) for g in (gamma0, gamma1, gamma2))
    be0, be1, be2 = (b.reshape(1, H) for b in (beta0, beta1, beta2))
    wc1 = Wc1.T
    bc1r = bc1.reshape(1, 32)
    wc2r = Wc2.reshape(1, 32)
    bc2r = bc2.reshape(1, 1)

    agg0 = _agg16(x_pad, src3, dst3)
    t0, s0, q0, rec = _layer0_tc(agg0[0], agg0[1], x_pad, wl0, bl0r, wr0)
    h1 = _bn_relu(t0, s0, q0, g0, be0)

    agg1 = _agg64(h1, src3, dst3)
    t1, s1, q1 = _layer12_tc(agg1, h1, rec, wl1, bl1r, wr1)
    h2 = _bn_relu(t1, s1, q1, g1, be1)

    agg2 = _agg64(h2, src3, dst3)
    t2, s2, q2 = _layer12_tc(agg2, h2, rec, wl2, bl2r, wr2)
    logits = _bn_head(t2, s2, q2, g2, be2, wc1, bc1r, wc2r, bc2r)
    return logits.reshape(N)
